# Initial kernel scaffold; baseline (speedup 1.0000x reference)
import jax
import jax.numpy as jnp
from jax.experimental import pallas as pl


def kernel(i, x, edges, graph_segments, W_in, b_in, W_edge, b_edge, W_filter, W_att, W_x, W_node, b_node, W_out, b_out, W1, b1, W2, b2):
    raise NotImplementedError("write your pallas kernel here")


# speedup vs baseline: 3.6977x; 3.6977x over previous
import functools

import jax
import jax.numpy as jnp
from jax import lax
from jax.experimental import pallas as pl
from jax.experimental.pallas import tpu as pltpu
from jax.experimental.pallas import tpu_sc as plsc

N = 50000
E = 800000
IN = 16
H = 64
HEADS = 4
K = 50
DEPTH = 6
NSEG = 1000
OUT = 64

NC = 2
NS = 16
NW = NC * NS
CH = 128
NCHUNK = 196
EPW = NCHUNK * CH
E_pad = NW * EPW
DUMP = N
ACC_ROWS = N + 8
NPS = N // NS
SEGCH = 13
NF_pad = NW * SEGCH * CH


def _silu(v):
    return v * jax.nn.sigmoid(v)


def _sc_gather2(table, src2, dst2, D):
    mesh = plsc.VectorSubcoreMesh(core_axis_name="c", subcore_axis_name="s")

    @functools.partial(
        pl.kernel,
        out_type=[jax.ShapeDtypeStruct((E_pad, D), jnp.float32),
                  jax.ShapeDtypeStruct((E_pad, D), jnp.float32)],
        mesh=mesh,
        compiler_params=pltpu.CompilerParams(use_tc_tiling_on_sc=False),
        scratch_types=[
            pltpu.VMEM((NCHUNK, CH), jnp.int32),
            pltpu.VMEM((NCHUNK, CH), jnp.int32),
            pltpu.VMEM((CH, D), jnp.float32),
            pltpu.VMEM((CH, D), jnp.float32),
            pltpu.SemaphoreType.DMA,
        ],
    )
    def k(table_h, src_h, dst_h, hs_h, hd_h, src_v, dst_v, ra, rb, sem):
        c = lax.axis_index("c")
        s = lax.axis_index("s")
        wid = s * NC + c
        base = wid * EPW
        pltpu.sync_copy(src_h.at[wid], src_v)
        pltpu.sync_copy(dst_h.at[wid], dst_v)

        def body(j, carry):
            ca = pltpu.async_copy(table_h.at[src_v.at[j]], ra, sem)
            cb = pltpu.async_copy(table_h.at[dst_v.at[j]], rb, sem)
            ca.wait()
            cb.wait()
            pltpu.sync_copy(ra, hs_h.at[pl.ds(base + j * CH, CH), :])
            pltpu.sync_copy(rb, hd_h.at[pl.ds(base + j * CH, CH), :])
            return carry

        lax.fori_loop(0, NCHUNK, body, 0)

    return k(table, src2, dst2)


def _sc_scatter(w64, o8e, dsc2):
    mesh = plsc.VectorSubcoreMesh(core_axis_name="c", subcore_axis_name="s")

    @functools.partial(
        pl.kernel,
        out_type=[jax.ShapeDtypeStruct((NC, N, H), jnp.float32),
                  jax.ShapeDtypeStruct((NC, N, 8), jnp.float32)],
        mesh=mesh,
        compiler_params=pltpu.CompilerParams(use_tc_tiling_on_sc=False),
        scratch_types=[
            pltpu.VMEM_SHARED((ACC_ROWS, 16), jnp.float32),
            pltpu.VMEM_SHARED((ACC_ROWS, 8), jnp.float32),
            pltpu.VMEM((NCHUNK, CH), jnp.int32),
            pltpu.VMEM((CH, 16), jnp.float32),
            pltpu.VMEM((CH, 8), jnp.float32),
            pltpu.VMEM((125, 16), jnp.float32),
        ],
    )
    def k(w_h, o8in_h, dst_h, o64_h, o8_h, acc16, acc8, dst_v, va, v8, zb):
        c = lax.axis_index("c")
        s = lax.axis_index("s")
        wid = s * NC + c
        ebase = wid * EPW

        def zrow(r, carry):
            zb[r, pl.ds(0, 16)] = jnp.zeros((16,), jnp.float32)
            return carry

        lax.fori_loop(0, 125, zrow, 0)
        pltpu.sync_copy(dst_h.at[wid], dst_v)

        def zk8(kk, cc):
            pltpu.sync_copy(zb.at[:, pl.ds(0, 8)],
                            acc8.at[pl.ds(s * NPS + kk * 125, 125), :])
            return cc

        lax.fori_loop(0, 25, zk8, 0)

        for g in range(4):
            def zk(kk, cc):
                pltpu.sync_copy(zb, acc16.at[pl.ds(s * NPS + kk * 125, 125), :])
                return cc

            lax.fori_loop(0, 25, zk, 0)
            plsc.subcore_barrier()

            def ga(j, carry, g=g):
                pltpu.sync_copy(
                    w_h.at[pl.ds(ebase + j * CH, CH), pl.ds(g * 16, 16)], va)
                pltpu.sync_copy(va, acc16.at[dst_v.at[j]], add=True)
                if g == 3:
                    pltpu.sync_copy(o8in_h.at[pl.ds(ebase + j * CH, CH), :], v8)
                    pltpu.sync_copy(v8, acc8.at[dst_v.at[j]], add=True)
                return carry

            lax.fori_loop(0, NCHUNK, ga, 0)
            plsc.subcore_barrier()
            pltpu.sync_copy(acc16.at[pl.ds(s * NPS, NPS), :],
                            o64_h.at[c, pl.ds(s * NPS, NPS), pl.ds(g * 16, 16)])
            plsc.subcore_barrier()
        pltpu.sync_copy(acc8.at[pl.ds(s * NPS, NPS), :],
                        o8_h.at[c, pl.ds(s * NPS, NPS), :])

    return k(w64, o8e, dsc2)


def _sc_segsum(hfinp, seg2):
    mesh = plsc.VectorSubcoreMesh(core_axis_name="c", subcore_axis_name="s")

    @functools.partial(
        pl.kernel,
        out_type=jax.ShapeDtypeStruct((NC, NSEG, OUT), jnp.float32),
        mesh=mesh,
        compiler_params=pltpu.CompilerParams(use_tc_tiling_on_sc=False),
        scratch_types=[
            pltpu.VMEM_SHARED((NSEG, OUT), jnp.float32),
            pltpu.VMEM((SEGCH, CH), jnp.int32),
            pltpu.VMEM((CH, OUT), jnp.float32),
            pltpu.VMEM((125, OUT), jnp.float32),
        ],
    )
    def k(hf_h, seg_h, o_h, accg, seg_v, vv, zb):
        c = lax.axis_index("c")
        s = lax.axis_index("s")
        wid = s * NC + c
        base = wid * SEGCH * CH

        def zrow(r, carry):
            for q in range(OUT // 16):
                zb[r, pl.ds(q * 16, 16)] = jnp.zeros((16,), jnp.float32)
            return carry

        lax.fori_loop(0, 125, zrow, 0)

        @pl.when(s < 8)
        def _():
            pltpu.sync_copy(zb, accg.at[pl.ds(s * 125, 125), :])

        plsc.subcore_barrier()
        pltpu.sync_copy(seg_h.at[wid], seg_v)

        def body(j, carry):
            pltpu.sync_copy(hf_h.at[pl.ds(base + j * CH, CH), :], vv)
            pltpu.sync_copy(vv, accg.at[seg_v.at[j]], add=True)
            return carry

        lax.fori_loop(0, SEGCH, body, 0)
        plsc.subcore_barrier()

        @pl.when(s < 8)
        def _():
            pltpu.sync_copy(accg.at[pl.ds(s * 125, 125), :],
                            o_h.at[c, pl.ds(s * 125, 125), :])

    return k(hfinp, seg2)


def _tc_embed(i_, W_in, b_in):
    B = 2000

    def body(i_ref, w_ref, b_ref, o_ref):
        o_ref[...] = _silu(
            jnp.dot(i_ref[...], w_ref[...], preferred_element_type=jnp.float32)
            + b_ref[...])

    return pl.pallas_call(
        body,
        grid=(N // B,),
        in_specs=[pl.BlockSpec((B, IN), lambda g: (g, 0)),
                  pl.BlockSpec((IN, H), lambda g: (0, 0)),
                  pl.BlockSpec((1, H), lambda g: (0, 0))],
        out_specs=pl.BlockSpec((B, H), lambda g: (g, 0)),
        out_shape=jax.ShapeDtypeStruct((N, H), jnp.float32),
    )(i_, W_in, b_in.reshape(1, H))


def _tc_geo(xs, xd):
    B = 2048

    def body(a_ref, b_ref, o_ref):
        r = b_ref[...] - a_ref[...]
        r4 = r[:, 0:4]
        lane = lax.broadcasted_iota(jnp.int32, (B, 4), 1)
        d2 = jnp.sum(jnp.where(lane < 3, r4 * r4, 0.0), axis=1, keepdims=True)
        d = jnp.sqrt(d2 + 1e-12)
        o_ref[...] = jnp.where(lane < 3, r4 / (d + 1e-5),
                               jnp.broadcast_to(d, (B, 4)))

    return pl.pallas_call(
        body,
        grid=(E_pad // B,),
        in_specs=[pl.BlockSpec((B, 16), lambda g: (g, 0)),
                  pl.BlockSpec((B, 16), lambda g: (g, 0))],
        out_specs=pl.BlockSpec((B, 4), lambda g: (g, 0)),
        out_shape=jax.ShapeDtypeStruct((E_pad, 4), jnp.float32),
    )(xs, xd)


def _tc_edge(hs, hd, dd4, We, be, Wf, Wa, Wx, Rexp, mu_row):
    B = 1024

    def body(hs_ref, hd_ref, dd_ref, we_ref, be_ref, wf_ref,
             wa_ref, wx_ref, rexp_ref, mu_ref, w64_ref, o8_ref):
        hsd = jnp.concatenate([hs_ref[...], hd_ref[...]], axis=1)
        z = (jnp.dot(hsd, we_ref[...], preferred_element_type=jnp.float32)
             + be_ref[...])
        dd = dd_ref[...]
        d = dd[:, 3:4]
        rbf = jnp.exp(-2.0 * (d - mu_ref[...]) ** 2)
        filt = _silu(jnp.dot(rbf, wf_ref[...], preferred_element_type=jnp.float32))
        he = _silu(z) * filt
        p4 = jnp.exp(jnp.dot(he, wa_ref[...], preferred_element_type=jnp.float32))
        p64 = jnp.dot(p4, rexp_ref[...], preferred_element_type=jnp.float32,
                      precision=lax.Precision.HIGHEST)
        w64_ref[...] = he * p64
        xc = jnp.dot(he, wx_ref[...], preferred_element_type=jnp.float32)
        o8_ref[...] = jnp.concatenate([p4, xc * dd], axis=1)

    return pl.pallas_call(
        body,
        grid=(E_pad // B,),
        in_specs=[pl.BlockSpec((B, H), lambda g: (g, 0)),
                  pl.BlockSpec((B, H), lambda g: (g, 0)),
                  pl.BlockSpec((B, 4), lambda g: (g, 0)),
                  pl.BlockSpec((2 * H, H), lambda g: (0, 0)),
                  pl.BlockSpec((1, H), lambda g: (0, 0)),
                  pl.BlockSpec((K, H), lambda g: (0, 0)),
                  pl.BlockSpec((H, HEADS), lambda g: (0, 0)),
                  pl.BlockSpec((H, 1), lambda g: (0, 0)),
                  pl.BlockSpec((HEADS, H), lambda g: (0, 0)),
                  pl.BlockSpec((1, K), lambda g: (0, 0))],
        out_specs=[pl.BlockSpec((B, H), lambda g: (g, 0)),
                   pl.BlockSpec((B, 8), lambda g: (g, 0))],
        out_shape=[jax.ShapeDtypeStruct((E_pad, H), jnp.float32),
                   jax.ShapeDtypeStruct((E_pad, 8), jnp.float32)],
    )(hs, hd, dd4, We, be, Wf, Wa, Wx, Rexp, mu_row)


def _tc_node(h, o64, o8n, Wn, bn, Rexp, Wo=None, bo=None):
    B = 1000
    last = Wo is not None

    def body(h_ref, a_ref, s8_ref, wn_ref, bn_ref,
             rexp_ref, *rest):
        if last:
            wo_ref, bo_ref, h_out, hf_out = rest
        else:
            (h_out,) = rest
        m_raw = jnp.sum(a_ref[...], axis=0)
        s8 = jnp.sum(s8_ref[...], axis=0)
        den4 = s8[:, 0:4]
        inv64 = jnp.dot(1.0 / (den4 + 1e-30), rexp_ref[...],
                        preferred_element_type=jnp.float32,
                        precision=lax.Precision.HIGHEST)
        m = m_raw * inv64
        lane8 = lax.broadcasted_iota(jnp.int32, (B, 8), 1)
        sp2 = jnp.sum(jnp.where((lane8 >= 4) & (lane8 < 7), s8 * s8, 0.0),
                      axis=1, keepdims=True)
        spn = jnp.sqrt(sp2 + 1e-12)
        hcur = h_ref[...]
        hmn = jnp.concatenate([hcur, m, spn], axis=1)
        upd = _silu(
            jnp.dot(hmn, wn_ref[...], preferred_element_type=jnp.float32)
            + bn_ref[...])
        hn = hcur + upd
        h_out[...] = hn
        if last:
            hf_out[...] = (jnp.dot(hn, wo_ref[...],
                                   preferred_element_type=jnp.float32)
                           + bo_ref[...])

    in_specs = [pl.BlockSpec((B, H), lambda g: (g, 0)),
                pl.BlockSpec((NC, B, H), lambda g: (0, g, 0)),
                pl.BlockSpec((NC, B, 8), lambda g: (0, g, 0)),
                pl.BlockSpec((2 * H + 1, H), lambda g: (0, 0)),
                pl.BlockSpec((1, H), lambda g: (0, 0)),
                pl.BlockSpec((HEADS, H), lambda g: (0, 0))]
    args = [h, o64, o8n, Wn, bn, Rexp]
    out_specs = [pl.BlockSpec((B, H), lambda g: (g, 0))]
    out_shape = [jax.ShapeDtypeStruct((N, H), jnp.float32)]
    if last:
        in_specs += [pl.BlockSpec((H, OUT), lambda g: (0, 0)),
                     pl.BlockSpec((1, OUT), lambda g: (0, 0))]
        args += [Wo, bo.reshape(1, OUT)]
        out_specs += [pl.BlockSpec((B, OUT), lambda g: (g, 0))]
        out_shape += [jax.ShapeDtypeStruct((N, OUT), jnp.float32)]

    res = pl.pallas_call(
        body,
        grid=(N // B,),
        in_specs=in_specs,
        out_specs=out_specs,
        out_shape=out_shape,
    )(*args)
    return res if last else res[0]


def _tc_mlp(y2, W1, b1, W2, b2):
    def body(y_ref, w1_ref, b1_ref, w2_ref, b2_ref, o_ref):
        q = jnp.sum(y_ref[...], axis=0)
        t = _silu(jnp.dot(q, w1_ref[...], preferred_element_type=jnp.float32)
                  + b1_ref[...])
        o_ref[...] = (jnp.dot(t, w2_ref[...], preferred_element_type=jnp.float32)
                      + b2_ref[...])

    return pl.pallas_call(
        body,
        grid=(1,),
        in_specs=[pl.BlockSpec((NC, NSEG, OUT), lambda g: (0, 0, 0)),
                  pl.BlockSpec((OUT, 64), lambda g: (0, 0)),
                  pl.BlockSpec((1, 64), lambda g: (0, 0)),
                  pl.BlockSpec((64, 1), lambda g: (0, 0)),
                  pl.BlockSpec((1, 1), lambda g: (0, 0))],
        out_specs=pl.BlockSpec((NSEG, 1), lambda g: (0, 0)),
        out_shape=jax.ShapeDtypeStruct((NSEG, 1), jnp.float32),
    )(y2, W1, b1.reshape(1, 64), W2, b2.reshape(1, 1))


def kernel(i, x, edges, graph_segments, W_in, b_in, W_edge, b_edge, W_filter,
           W_att, W_x, W_node, b_node, W_out, b_out, W1, b1, W2, b2):
    f32 = jnp.float32
    src = edges[0].astype(jnp.int32)
    dst = edges[1].astype(jnp.int32)
    padlen = E_pad - E
    src2 = jnp.pad(src, (0, padlen)).reshape(NW, NCHUNK, CH)
    dst2 = jnp.pad(dst, (0, padlen)).reshape(NW, NCHUNK, CH)
    dsc2 = jnp.pad(dst, (0, padlen), constant_values=DUMP).reshape(NW, NCHUNK, CH)

    h = _tc_embed(i.astype(f32), W_in, b_in)
    xpad = jnp.pad(x.astype(f32), ((0, 0), (0, 13)))
    xs, xd = _sc_gather2(xpad, src2, dst2, 16)
    dd4 = _tc_geo(xs, xd)

    Rexp = jnp.repeat(jnp.eye(HEADS, dtype=f32), H // HEADS, axis=1)
    mu_row = jnp.linspace(0.0, 5.0, K).reshape(1, K).astype(f32)

    hfin = None
    for l in range(DEPTH):
        hs, hd = _sc_gather2(h, src2, dst2, H)
        w64, o8e = _tc_edge(hs, hd, dd4, W_edge[l],
                            b_edge[l].reshape(1, H), W_filter[l], W_att[l],
                            W_x[l], Rexp, mu_row)
        o64, o8n = _sc_scatter(w64, o8e, dsc2)
        if l < DEPTH - 1:
            h = _tc_node(h, o64, o8n, W_node[l], b_node[l].reshape(1, H),
                         Rexp)
        else:
            h, hfin = _tc_node(h, o64, o8n, W_node[l],
                               b_node[l].reshape(1, H), Rexp, W_out, b_out)

    hfp = jnp.pad(hfin, ((0, NF_pad - N), (0, 0)))
    seg2 = jnp.pad(graph_segments.astype(jnp.int32), (0, NF_pad - N),
                   constant_values=NSEG - 1).reshape(NW, SEGCH, CH)
    y2 = _sc_segsum(hfp, seg2)
    return _tc_mlp(y2, W1, b1, W2, b2)


# speedup vs baseline: 4.4666x; 1.2079x over previous
import functools

import jax
import jax.numpy as jnp
from jax import lax
from jax.experimental import pallas as pl
from jax.experimental.pallas import tpu as pltpu
from jax.experimental.pallas import tpu_sc as plsc

N = 50000
E = 800000
IN = 16
H = 64
HEADS = 4
K = 50
DEPTH = 6
NSEG = 1000
OUT = 64

NC = 2
NS = 16
NW = NC * NS
CH = 128
NCHUNK = 196
EPW = NCHUNK * CH
E_pad = NW * EPW
DUMP = N
ACC_ROWS = N + 8
NPS = N // NS
SEGCH = 13
NF_pad = NW * SEGCH * CH


def _silu(v):
    return v * jax.nn.sigmoid(v)


def _sc_gather2(table, src2, dst2, D):
    mesh = plsc.VectorSubcoreMesh(core_axis_name="c", subcore_axis_name="s")

    NSLOT = 4
    SUP = NCHUNK // NSLOT

    @functools.partial(
        pl.kernel,
        out_type=[jax.ShapeDtypeStruct((E_pad, D), jnp.float32),
                  jax.ShapeDtypeStruct((E_pad, D), jnp.float32)],
        mesh=mesh,
        compiler_params=pltpu.CompilerParams(use_tc_tiling_on_sc=False),
        scratch_types=[
            pltpu.VMEM((NCHUNK, CH), jnp.int32),
            pltpu.VMEM((NCHUNK, CH), jnp.int32),
            pltpu.VMEM((NSLOT, CH, D), jnp.float32),
            pltpu.VMEM((NSLOT, CH, D), jnp.float32),
            pltpu.SemaphoreType.DMA((NSLOT,)),
            pltpu.SemaphoreType.DMA((NSLOT,)),
            pltpu.SemaphoreType.DMA((NSLOT,)),
            pltpu.SemaphoreType.DMA((NSLOT,)),
        ],
    )
    def k(table_h, src_h, dst_h, hs_h, hd_h, src_v, dst_v, ra, rb,
          sga, sgb, swa, swb):
        c = lax.axis_index("c")
        s = lax.axis_index("s")
        wid = s * NC + c
        base = wid * EPW
        pltpu.sync_copy(src_h.at[wid], src_v)
        pltpu.sync_copy(dst_h.at[wid], dst_v)

        def issue_g(j, i):
            pltpu.async_copy(table_h.at[src_v.at[j]], ra.at[i], sga.at[i])
            pltpu.async_copy(table_h.at[dst_v.at[j]], rb.at[i], sgb.at[i])

        for i in range(NSLOT):
            issue_g(i, i)

        def body(t, carry):
            for i in range(NSLOT):
                j = t * NSLOT + i
                pltpu.make_async_copy(hs_h.at[pl.ds(0, CH), :], ra.at[i],
                                      sga.at[i]).wait()
                pltpu.make_async_copy(hs_h.at[pl.ds(0, CH), :], rb.at[i],
                                      sgb.at[i]).wait()
                pltpu.async_copy(ra.at[i], hs_h.at[pl.ds(base + j * CH, CH), :],
                                 swa.at[i])
                pltpu.async_copy(rb.at[i], hd_h.at[pl.ds(base + j * CH, CH), :],
                                 swb.at[i])
            for i in range(NSLOT):
                jn = (t + 1) * NSLOT + i
                pltpu.make_async_copy(hs_h.at[pl.ds(0, CH), :], ra.at[i],
                                      swa.at[i]).wait()
                pltpu.make_async_copy(hs_h.at[pl.ds(0, CH), :], rb.at[i],
                                      swb.at[i]).wait()

                @pl.when(t < SUP - 1)
                def _(jn=jn, i=i):
                    issue_g(jn, i)

            return carry

        lax.fori_loop(0, SUP, body, 0)

    return k(table, src2, dst2)


def _sc_scatter(w64, o8e, dsc2):
    mesh = plsc.VectorSubcoreMesh(core_axis_name="c", subcore_axis_name="s")

    @functools.partial(
        pl.kernel,
        out_type=[jax.ShapeDtypeStruct((NC, N, H), jnp.float32),
                  jax.ShapeDtypeStruct((NC, N, 8), jnp.float32)],
        mesh=mesh,
        compiler_params=pltpu.CompilerParams(use_tc_tiling_on_sc=False),
        scratch_types=[
            pltpu.VMEM_SHARED((ACC_ROWS, 16), jnp.float32),
            pltpu.VMEM_SHARED((ACC_ROWS, 8), jnp.float32),
            pltpu.VMEM((NCHUNK, CH), jnp.int32),
            pltpu.VMEM((4, CH, 16), jnp.float32),
            pltpu.VMEM((4, CH, 8), jnp.float32),
            pltpu.VMEM((625, 16), jnp.float32),
            pltpu.SemaphoreType.DMA((4,)),
            pltpu.SemaphoreType.DMA((4,)),
            pltpu.SemaphoreType.DMA((4,)),
            pltpu.SemaphoreType.DMA((4,)),
        ],
    )
    def k(w_h, o8in_h, dst_h, o64_h, o8_h, acc16, acc8, dst_v, va, v8, zb,
          sl, ss, sl8, ss8):
        c = lax.axis_index("c")
        s = lax.axis_index("s")
        wid = s * NC + c
        ebase = wid * EPW
        SUP = NCHUNK // 4

        def zrow(r, carry):
            zb[r, pl.ds(0, 16)] = jnp.zeros((16,), jnp.float32)
            return carry

        lax.fori_loop(0, 625, zrow, 0)
        pltpu.sync_copy(dst_h.at[wid], dst_v)

        def zk8(kk, cc):
            pltpu.sync_copy(zb.at[pl.ds(0, 125), pl.ds(0, 8)],
                            acc8.at[pl.ds(s * NPS + kk * 125, 125), :])
            return cc

        lax.fori_loop(0, 25, zk8, 0)

        for g in range(4):
            def zk(kk, cc):
                pltpu.sync_copy(zb, acc16.at[pl.ds(s * NPS + kk * 625, 625), :])
                return cc

            lax.fori_loop(0, 5, zk, 0)
            plsc.subcore_barrier()

            def issue_l(j, i, g=g):
                pltpu.async_copy(
                    w_h.at[pl.ds(ebase + j * CH, CH), pl.ds(g * 16, 16)],
                    va.at[i], sl.at[i])
                if g == 3:
                    pltpu.async_copy(o8in_h.at[pl.ds(ebase + j * CH, CH), :],
                                     v8.at[i], sl8.at[i])

            for i in range(4):
                issue_l(i, i)

            def ga(t, carry, g=g, issue_l=issue_l):
                for i in range(4):
                    j = t * 4 + i
                    pltpu.make_async_copy(
                        w_h.at[pl.ds(0, CH), pl.ds(0, 16)], va.at[i],
                        sl.at[i]).wait()
                    pltpu.async_copy(va.at[i], acc16.at[dst_v.at[j]],
                                     ss.at[i], add=True)
                    if g == 3:
                        pltpu.make_async_copy(
                            o8in_h.at[pl.ds(0, CH), :], v8.at[i],
                            sl8.at[i]).wait()
                        pltpu.async_copy(v8.at[i], acc8.at[dst_v.at[j]],
                                         ss8.at[i], add=True)
                for i in range(4):
                    jn = (t + 1) * 4 + i
                    pltpu.make_async_copy(
                        w_h.at[pl.ds(0, CH), pl.ds(0, 16)], va.at[i],
                        ss.at[i]).wait()
                    if g == 3:
                        pltpu.make_async_copy(
                            o8in_h.at[pl.ds(0, CH), :], v8.at[i],
                            ss8.at[i]).wait()

                    @pl.when(t < SUP - 1)
                    def _(jn=jn, i=i):
                        issue_l(jn, i)

                return carry

            lax.fori_loop(0, SUP, ga, 0)
            plsc.subcore_barrier()
            pltpu.sync_copy(acc16.at[pl.ds(s * NPS, NPS), :],
                            o64_h.at[c, pl.ds(s * NPS, NPS), pl.ds(g * 16, 16)])
            plsc.subcore_barrier()
        pltpu.sync_copy(acc8.at[pl.ds(s * NPS, NPS), :],
                        o8_h.at[c, pl.ds(s * NPS, NPS), :])

    return k(w64, o8e, dsc2)


def _sc_segsum(hfinp, seg2):
    mesh = plsc.VectorSubcoreMesh(core_axis_name="c", subcore_axis_name="s")

    @functools.partial(
        pl.kernel,
        out_type=jax.ShapeDtypeStruct((NC, NSEG, OUT), jnp.float32),
        mesh=mesh,
        compiler_params=pltpu.CompilerParams(use_tc_tiling_on_sc=False),
        scratch_types=[
            pltpu.VMEM_SHARED((NSEG, OUT), jnp.float32),
            pltpu.VMEM((SEGCH, CH), jnp.int32),
            pltpu.VMEM((CH, OUT), jnp.float32),
            pltpu.VMEM((125, OUT), jnp.float32),
        ],
    )
    def k(hf_h, seg_h, o_h, accg, seg_v, vv, zb):
        c = lax.axis_index("c")
        s = lax.axis_index("s")
        wid = s * NC + c
        base = wid * SEGCH * CH

        def zrow(r, carry):
            for q in range(OUT // 16):
                zb[r, pl.ds(q * 16, 16)] = jnp.zeros((16,), jnp.float32)
            return carry

        lax.fori_loop(0, 125, zrow, 0)

        @pl.when(s < 8)
        def _():
            pltpu.sync_copy(zb, accg.at[pl.ds(s * 125, 125), :])

        plsc.subcore_barrier()
        pltpu.sync_copy(seg_h.at[wid], seg_v)

        def body(j, carry):
            pltpu.sync_copy(hf_h.at[pl.ds(base + j * CH, CH), :], vv)
            pltpu.sync_copy(vv, accg.at[seg_v.at[j]], add=True)
            return carry

        lax.fori_loop(0, SEGCH, body, 0)
        plsc.subcore_barrier()

        @pl.when(s < 8)
        def _():
            pltpu.sync_copy(accg.at[pl.ds(s * 125, 125), :],
                            o_h.at[c, pl.ds(s * 125, 125), :])

    return k(hfinp, seg2)


def _tc_embed(i_, W_in, b_in):
    B = 2000

    def body(i_ref, w_ref, b_ref, o_ref):
        o_ref[...] = _silu(
            jnp.dot(i_ref[...], w_ref[...], preferred_element_type=jnp.float32)
            + b_ref[...])

    return pl.pallas_call(
        body,
        grid=(N // B,),
        in_specs=[pl.BlockSpec((B, IN), lambda g: (g, 0)),
                  pl.BlockSpec((IN, H), lambda g: (0, 0)),
                  pl.BlockSpec((1, H), lambda g: (0, 0))],
        out_specs=pl.BlockSpec((B, H), lambda g: (g, 0)),
        out_shape=jax.ShapeDtypeStruct((N, H), jnp.float32),
    )(i_, W_in, b_in.reshape(1, H))


def _tc_geo(xs, xd):
    B = 2048

    def body(a_ref, b_ref, o_ref):
        r = b_ref[...] - a_ref[...]
        r4 = r[:, 0:4]
        lane = lax.broadcasted_iota(jnp.int32, (B, 4), 1)
        d2 = jnp.sum(jnp.where(lane < 3, r4 * r4, 0.0), axis=1, keepdims=True)
        d = jnp.sqrt(d2 + 1e-12)
        o_ref[...] = jnp.where(lane < 3, r4 / (d + 1e-5),
                               jnp.broadcast_to(d, (B, 4)))

    return pl.pallas_call(
        body,
        grid=(E_pad // B,),
        in_specs=[pl.BlockSpec((B, 16), lambda g: (g, 0)),
                  pl.BlockSpec((B, 16), lambda g: (g, 0))],
        out_specs=pl.BlockSpec((B, 4), lambda g: (g, 0)),
        out_shape=jax.ShapeDtypeStruct((E_pad, 4), jnp.float32),
    )(xs, xd)


def _tc_edge(hs, hd, dd4, We, be, Wf, Wa, Wx, Rexp, mu_row):
    B = 1024

    def body(hs_ref, hd_ref, dd_ref, we_ref, be_ref, wf_ref,
             wa_ref, wx_ref, rexp_ref, mu_ref, w64_ref, o8_ref):
        hsd = jnp.concatenate([hs_ref[...], hd_ref[...]], axis=1)
        z = (jnp.dot(hsd, we_ref[...], preferred_element_type=jnp.float32)
             + be_ref[...])
        dd = dd_ref[...]
        d = dd[:, 3:4]
        rbf = jnp.exp(-2.0 * (d - mu_ref[...]) ** 2)
        filt = _silu(jnp.dot(rbf, wf_ref[...], preferred_element_type=jnp.float32))
        he = _silu(z) * filt
        p4 = jnp.exp(jnp.dot(he, wa_ref[...], preferred_element_type=jnp.float32))
        p64 = jnp.dot(p4, rexp_ref[...], preferred_element_type=jnp.float32,
                      precision=lax.Precision.HIGHEST)
        w64_ref[...] = he * p64
        xc = jnp.dot(he, wx_ref[...], preferred_element_type=jnp.float32)
        o8_ref[...] = jnp.concatenate([p4, xc * dd], axis=1)

    return pl.pallas_call(
        body,
        grid=(E_pad // B,),
        in_specs=[pl.BlockSpec((B, H), lambda g: (g, 0)),
                  pl.BlockSpec((B, H), lambda g: (g, 0)),
                  pl.BlockSpec((B, 4), lambda g: (g, 0)),
                  pl.BlockSpec((2 * H, H), lambda g: (0, 0)),
                  pl.BlockSpec((1, H), lambda g: (0, 0)),
                  pl.BlockSpec((K, H), lambda g: (0, 0)),
                  pl.BlockSpec((H, HEADS), lambda g: (0, 0)),
                  pl.BlockSpec((H, 1), lambda g: (0, 0)),
                  pl.BlockSpec((HEADS, H), lambda g: (0, 0)),
                  pl.BlockSpec((1, K), lambda g: (0, 0))],
        out_specs=[pl.BlockSpec((B, H), lambda g: (g, 0)),
                   pl.BlockSpec((B, 8), lambda g: (g, 0))],
        out_shape=[jax.ShapeDtypeStruct((E_pad, H), jnp.float32),
                   jax.ShapeDtypeStruct((E_pad, 8), jnp.float32)],
    )(hs, hd, dd4, We, be, Wf, Wa, Wx, Rexp, mu_row)


def _tc_node(h, o64, o8n, Wn, bn, Rexp, Wo=None, bo=None):
    B = 1000
    last = Wo is not None

    def body(h_ref, a_ref, s8_ref, wn_ref, bn_ref,
             rexp_ref, *rest):
        if last:
            wo_ref, bo_ref, h_out, hf_out = rest
        else:
            (h_out,) = rest
        m_raw = jnp.sum(a_ref[...], axis=0)
        s8 = jnp.sum(s8_ref[...], axis=0)
        den4 = s8[:, 0:4]
        inv64 = jnp.dot(1.0 / (den4 + 1e-30), rexp_ref[...],
                        preferred_element_type=jnp.float32,
                        precision=lax.Precision.HIGHEST)
        m = m_raw * inv64
        lane8 = lax.broadcasted_iota(jnp.int32, (B, 8), 1)
        sp2 = jnp.sum(jnp.where((lane8 >= 4) & (lane8 < 7), s8 * s8, 0.0),
                      axis=1, keepdims=True)
        spn = jnp.sqrt(sp2 + 1e-12)
        hcur = h_ref[...]
        hmn = jnp.concatenate([hcur, m, spn], axis=1)
        upd = _silu(
            jnp.dot(hmn, wn_ref[...], preferred_element_type=jnp.float32)
            + bn_ref[...])
        hn = hcur + upd
        h_out[...] = hn
        if last:
            hf_out[...] = (jnp.dot(hn, wo_ref[...],
                                   preferred_element_type=jnp.float32)
                           + bo_ref[...])

    in_specs = [pl.BlockSpec((B, H), lambda g: (g, 0)),
                pl.BlockSpec((NC, B, H), lambda g: (0, g, 0)),
                pl.BlockSpec((NC, B, 8), lambda g: (0, g, 0)),
                pl.BlockSpec((2 * H + 1, H), lambda g: (0, 0)),
                pl.BlockSpec((1, H), lambda g: (0, 0)),
                pl.BlockSpec((HEADS, H), lambda g: (0, 0))]
    args = [h, o64, o8n, Wn, bn, Rexp]
    out_specs = [pl.BlockSpec((B, H), lambda g: (g, 0))]
    out_shape = [jax.ShapeDtypeStruct((N, H), jnp.float32)]
    if last:
        in_specs += [pl.BlockSpec((H, OUT), lambda g: (0, 0)),
                     pl.BlockSpec((1, OUT), lambda g: (0, 0))]
        args += [Wo, bo.reshape(1, OUT)]
        out_specs += [pl.BlockSpec((B, OUT), lambda g: (g, 0))]
        out_shape += [jax.ShapeDtypeStruct((N, OUT), jnp.float32)]

    res = pl.pallas_call(
        body,
        grid=(N // B,),
        in_specs=in_specs,
        out_specs=out_specs,
        out_shape=out_shape,
    )(*args)
    return res if last else res[0]


def _tc_mlp(y2, W1, b1, W2, b2):
    def body(y_ref, w1_ref, b1_ref, w2_ref, b2_ref, o_ref):
        q = jnp.sum(y_ref[...], axis=0)
        t = _silu(jnp.dot(q, w1_ref[...], preferred_element_type=jnp.float32)
                  + b1_ref[...])
        o_ref[...] = (jnp.dot(t, w2_ref[...], preferred_element_type=jnp.float32)
                      + b2_ref[...])

    return pl.pallas_call(
        body,
        grid=(1,),
        in_specs=[pl.BlockSpec((NC, NSEG, OUT), lambda g: (0, 0, 0)),
                  pl.BlockSpec((OUT, 64), lambda g: (0, 0)),
                  pl.BlockSpec((1, 64), lambda g: (0, 0)),
                  pl.BlockSpec((64, 1), lambda g: (0, 0)),
                  pl.BlockSpec((1, 1), lambda g: (0, 0))],
        out_specs=pl.BlockSpec((NSEG, 1), lambda g: (0, 0)),
        out_shape=jax.ShapeDtypeStruct((NSEG, 1), jnp.float32),
    )(y2, W1, b1.reshape(1, 64), W2, b2.reshape(1, 1))


def kernel(i, x, edges, graph_segments, W_in, b_in, W_edge, b_edge, W_filter,
           W_att, W_x, W_node, b_node, W_out, b_out, W1, b1, W2, b2):
    f32 = jnp.float32
    src = edges[0].astype(jnp.int32)
    dst = edges[1].astype(jnp.int32)
    padlen = E_pad - E
    src2 = jnp.pad(src, (0, padlen)).reshape(NW, NCHUNK, CH)
    dst2 = jnp.pad(dst, (0, padlen)).reshape(NW, NCHUNK, CH)
    dsc2 = jnp.pad(dst, (0, padlen), constant_values=DUMP).reshape(NW, NCHUNK, CH)

    h = _tc_embed(i.astype(f32), W_in, b_in)
    xpad = jnp.pad(x.astype(f32), ((0, 0), (0, 13)))
    xs, xd = _sc_gather2(xpad, src2, dst2, 16)
    dd4 = _tc_geo(xs, xd)

    Rexp = jnp.repeat(jnp.eye(HEADS, dtype=f32), H // HEADS, axis=1)
    mu_row = jnp.linspace(0.0, 5.0, K).reshape(1, K).astype(f32)

    hfin = None
    for l in range(DEPTH):
        hs, hd = _sc_gather2(h, src2, dst2, H)
        w64, o8e = _tc_edge(hs, hd, dd4, W_edge[l],
                            b_edge[l].reshape(1, H), W_filter[l], W_att[l],
                            W_x[l], Rexp, mu_row)
        o64, o8n = _sc_scatter(w64, o8e, dsc2)
        if l < DEPTH - 1:
            h = _tc_node(h, o64, o8n, W_node[l], b_node[l].reshape(1, H),
                         Rexp)
        else:
            h, hfin = _tc_node(h, o64, o8n, W_node[l],
                               b_node[l].reshape(1, H), Rexp, W_out, b_out)

    hfp = jnp.pad(hfin, ((0, NF_pad - N), (0, 0)))
    seg2 = jnp.pad(graph_segments.astype(jnp.int32), (0, NF_pad - N),
                   constant_values=NSEG - 1).reshape(NW, SEGCH, CH)
    y2 = _sc_segsum(hfp, seg2)
    return _tc_mlp(y2, W1, b1, W2, b2)
